# Initial kernel scaffold; baseline (speedup 1.0000x reference)
import jax
import jax.numpy as jnp
from jax.experimental import pallas as pl


def kernel(x, index, dim_size, W):
    raise NotImplementedError("write your pallas kernel here")


# speedup vs baseline: 6.7368x; 6.7368x over previous
import functools

import jax
import jax.numpy as jnp
from jax import lax
from jax.experimental import pallas as pl
from jax.experimental.pallas import tpu as pltpu
from jax.experimental.pallas import tpu_sc as plsc

N_EDGES = 320000
N_NODES = 10000
D = 128
AVG_DEG_LOG = 3.5

NW = 32
NPW = 320
EB = 256
RB = 16
AGG_W = 5 * D
NV = D // 16


def _sc_body(x_hbm, idx_hbm, cuts_hbm, agg_hbm, cuts_v, ibuf, xbuf, obuf,
             sem0, sem1):
    c = lax.axis_index("c")
    s = lax.axis_index("s")
    wid = s * 2 + c
    pltpu.sync_copy(cuts_hbm, cuts_v)
    cv = cuts_v[pl.ds(wid, 16)]
    e0 = cv[0]
    e1 = cv[1]
    n0 = wid * NPW
    n1 = jnp.minimum(n0 + NPW, N_NODES)

    zero = jnp.zeros((16,), jnp.float32)
    pinf = jnp.full((16,), jnp.inf, jnp.float32)
    ninf = jnp.full((16,), -jnp.inf, jnp.float32)

    def fresh_acc():
        return ([zero] * NV, [zero] * NV, [pinf] * NV, [ninf] * NV)

    def flush_body(st):
        n, cnt, sv, qv, mnv, mxv = st
        k = n - n0
        bi = k // RB
        buf = lax.rem(bi, 2)
        slot = lax.rem(k, RB)

        @pl.when((slot == 0) & (bi >= 2))
        def _():
            prev = (n - 2 * RB) * AGG_W

            @pl.when(buf == 0)
            def _():
                pltpu.make_async_copy(
                    obuf.at[pl.ds(0, RB * AGG_W)],
                    agg_hbm.at[pl.ds(prev, RB * AGG_W)], sem0).wait()

            @pl.when(buf == 1)
            def _():
                pltpu.make_async_copy(
                    obuf.at[pl.ds(RB * AGG_W, RB * AGG_W)],
                    agg_hbm.at[pl.ds(prev, RB * AGG_W)], sem1).wait()

        base = (buf * RB + slot) * AGG_W
        cnt_v = jnp.full((16,), cnt, jnp.float32)
        for j in range(NV):
            obuf[pl.ds(base + j * 16, 16)] = sv[j]
            obuf[pl.ds(base + D + j * 16, 16)] = qv[j]
            obuf[pl.ds(base + 2 * D + j * 16, 16)] = mnv[j]
            obuf[pl.ds(base + 3 * D + j * 16, 16)] = mxv[j]
            obuf[pl.ds(base + 4 * D + j * 16, 16)] = cnt_v

        @pl.when(slot == RB - 1)
        def _():
            dst = (n - (RB - 1)) * AGG_W

            @pl.when(buf == 0)
            def _():
                pltpu.async_copy(
                    obuf.at[pl.ds(0, RB * AGG_W)],
                    agg_hbm.at[pl.ds(dst, RB * AGG_W)], sem0)

            @pl.when(buf == 1)
            def _():
                pltpu.async_copy(
                    obuf.at[pl.ds(RB * AGG_W, RB * AGG_W)],
                    agg_hbm.at[pl.ds(dst, RB * AGG_W)], sem1)

        sv, qv, mnv, mxv = fresh_acc()
        return (n + 1, jnp.zeros((), jnp.float32), sv, qv, mnv, mxv)

    def edge_body(e_loc, st):
        eid = ibuf[pl.ds(e_loc, 16)][0]
        st = lax.fori_loop(st[0], eid, lambda m, t: flush_body(t), st)
        n, cnt, sv, qv, mnv, mxv = st
        base = e_loc * D
        nsv, nqv, nmn, nmx = [], [], [], []
        for j in range(NV):
            xv = xbuf[pl.ds(base + j * 16, 16)]
            nsv.append(sv[j] + xv)
            nqv.append(qv[j] + xv * xv)
            nmn.append(jnp.minimum(mnv[j], xv))
            nmx.append(jnp.maximum(mxv[j], xv))
        return (n, cnt + 1.0, nsv, nqv, nmn, nmx)

    def blk_body(b, st):
        pltpu.sync_copy(x_hbm.at[pl.ds(b * EB * D, EB * D)], xbuf)
        pltpu.sync_copy(idx_hbm.at[pl.ds(b * EB, EB)], ibuf.at[pl.ds(0, EB)])
        lo = jnp.maximum(b * EB, e0) - b * EB
        hi = jnp.minimum((b + 1) * EB, e1) - b * EB
        return lax.fori_loop(lo, hi, edge_body, st)

    sv, qv, mnv, mxv = fresh_acc()
    st = (n0, jnp.zeros((), jnp.float32), sv, qv, mnv, mxv)
    b0 = e0 // EB
    b1 = (e1 + EB - 1) // EB
    st = lax.fori_loop(b0, b1, blk_body, st)
    st = lax.fori_loop(st[0], n1, lambda m, t: flush_body(t), st)

    pltpu.make_async_copy(
        obuf.at[pl.ds(0, RB * AGG_W)],
        agg_hbm.at[pl.ds((n1 - 2 * RB) * AGG_W, RB * AGG_W)], sem0).wait()
    pltpu.make_async_copy(
        obuf.at[pl.ds(RB * AGG_W, RB * AGG_W)],
        agg_hbm.at[pl.ds((n1 - RB) * AGG_W, RB * AGG_W)], sem1).wait()


_sc_reduce = pl.kernel(
    _sc_body,
    out_type=jax.ShapeDtypeStruct((N_NODES * AGG_W,), jnp.float32),
    mesh=plsc.VectorSubcoreMesh(core_axis_name="c", subcore_axis_name="s"),
    scratch_types=[
        pltpu.VMEM((48,), jnp.int32),
        pltpu.VMEM((EB + 16,), jnp.int32),
        pltpu.VMEM((EB * D,), jnp.float32),
        pltpu.VMEM((2 * RB * AGG_W,), jnp.float32),
        pltpu.SemaphoreType.DMA,
        pltpu.SemaphoreType.DMA,
    ],
)


def _tc_body(agg_ref, w_ref, out_ref):
    a = agg_ref[...]
    sm = a[:, 0:D]
    sq = a[:, D:2 * D]
    mn = a[:, 2 * D:3 * D]
    mx = a[:, 3 * D:4 * D]
    deg = a[:, 4 * D:5 * D]
    empty = deg == 0.0
    degs = jnp.maximum(deg, 1.0)
    mean = sm / degs
    var = sq / degs - mean * mean
    std = jnp.sqrt(jnp.maximum(var, 0.0) + 1e-5)
    mn = jnp.where(empty, 0.0, mn)
    mx = jnp.where(empty, 0.0, mx)
    ld = jnp.log(deg + 1.0)
    amp = ld / AVG_DEG_LOG
    att = jnp.where(empty, 1.0, AVG_DEG_LOG / jnp.where(empty, 1.0, ld))

    def coef(k):
        return (w_ref[0, 3 * k] + w_ref[0, 3 * k + 1] * amp
                + w_ref[0, 3 * k + 2] * att)

    out_ref[...] = (mean * coef(0) + mn * coef(1) + mx * coef(2)
                    + std * coef(3))


def _tc_finish(agg, W):
    rows = 1000
    return pl.pallas_call(
        _tc_body,
        grid=(N_NODES // rows,),
        in_specs=[
            pl.BlockSpec((rows, AGG_W), lambda i: (i, 0)),
            pl.BlockSpec(memory_space=pltpu.SMEM),
        ],
        out_specs=pl.BlockSpec((rows, D), lambda i: (i, 0)),
        out_shape=jax.ShapeDtypeStruct((N_NODES, D), jnp.float32),
    )(agg, W)


def kernel(x, index, dim_size, W):
    del dim_size
    bounds = jnp.minimum(
        jnp.arange(NW + 1, dtype=jnp.int32) * NPW, N_NODES)
    cuts = jnp.searchsorted(index, bounds, side="left").astype(jnp.int32)
    cuts = jnp.concatenate([cuts, jnp.zeros((48 - NW - 1,), jnp.int32)])
    agg = _sc_reduce(x.reshape(-1), index, cuts)
    return _tc_finish(agg.reshape(N_NODES, AGG_W), W)
